# baseline (speedup 1.0000x reference)
import functools

import jax
import jax.numpy as jnp
from jax import lax
from jax.experimental import pallas as pl
from jax.experimental.pallas import tpu as pltpu
from jax.experimental.pallas import tpu_sc as plsc

N_HID = 128
MAX_LEN = 240
LANES = 16
CHUNK = 128
MAX_PW = 79


def _proj_table_body(emb_ref, w_ref, b_ref, out_ref):
    p = lax.dot_general(
        emb_ref[...], w_ref[...],
        dimension_numbers=(((1,), (1,)), ((), ())),
        preferred_element_type=jnp.float32,
    )
    out_ref[...] = p + b_ref[...]


@functools.cache
def _make_sc_kernel(n_rows):
    n_chunks = n_rows // CHUNK
    info = plsc.get_sparse_core_info()
    nc, ns = info.num_cores, info.num_subcores
    nw = nc * ns

    mesh = plsc.VectorSubcoreMesh(core_axis_name="c", subcore_axis_name="s")

    @functools.partial(
        pl.kernel,
        mesh=mesh,
        out_type=jax.ShapeDtypeStruct((n_rows, N_HID), jnp.float32),
        scratch_types=(
            [pltpu.VMEM((CHUNK,), jnp.int32) for _ in range(4)]
            + [pltpu.VMEM((CHUNK, N_HID), jnp.float32) for _ in range(4)]
            + [pltpu.VMEM((CHUNK, N_HID), jnp.float32) for _ in range(3)]
            + [pltpu.VMEM_SHARED((MAX_LEN, N_HID), jnp.float32)]
            + [pltpu.SemaphoreType.DMA for _ in range(15)]
        ),
    )
    def sc_fn(x_hbm, t_hbm, p_hbm, out_hbm,
              iv0, iv1, iv2, iv3, xv0, xv1, xv2, xv3, ev0, ev1, ev2, p_sh,
              si0, si1, si2, si3, sx0, sx1, sx2, sx3,
              sg0, sg1, sg2, so0, so1, so2, so3):
        wid = lax.axis_index("s") * nc + lax.axis_index("c")

        iv = (iv0, iv1, iv2, iv3)
        si = (si0, si1, si2, si3)
        xv = (xv0, xv1, xv2, xv3)
        sx = (sx0, sx1, sx2, sx3)
        so = (so0, so1, so2, so3)
        ev = (ev0, ev1, ev2)
        sg = (sg0, sg1, sg2)

        @pl.when(lax.axis_index("s") == 0)
        def _():
            pltpu.sync_copy(p_hbm, p_sh)
        plsc.subcore_barrier()

        def valid(m):
            return wid + m * nw < n_chunks

        def row_base(m):
            return (wid + m * nw) * CHUNK

        def issue_idx(m, s4):
            @pl.when(valid(m))
            def _():
                pltpu.async_copy(
                    t_hbm.at[pl.ds(row_base(m), CHUNK)], iv[s4], si[s4])

        def issue_xg(m, s4, s4x, s3e):
            @pl.when(valid(m))
            def _():
                pltpu.make_async_copy(
                    t_hbm.at[pl.ds(row_base(m), CHUNK)], iv[s4], si[s4]).wait()

                @pl.when(m >= 4)
                def _():
                    pltpu.make_async_copy(
                        xv[s4x], out_hbm.at[pl.ds(0, CHUNK)], so[s4x]).wait()

                pltpu.async_copy(
                    x_hbm.at[pl.ds(row_base(m), CHUNK)], xv[s4x], sx[s4x])
                pltpu.async_copy(p_sh.at[iv[s4]], ev[s3e], sg[s3e])

        def crunch(m, s4x, s3e):
            @pl.when(valid(m))
            def _():
                rb = row_base(m)
                pltpu.make_async_copy(
                    x_hbm.at[pl.ds(rb, CHUNK)], xv[s4x], sx[s4x]).wait()
                pltpu.make_async_copy(
                    p_sh.at[iv[s4x]], ev[s3e], sg[s3e]).wait()

                def row_body(i, c):
                    for r in range(4):
                        for j in range(N_HID // LANES):
                            sl = pl.ds(j * LANES, LANES)
                            plsc.addupdate(xv[s4x].at[4 * i + r, sl],
                                           ev[s3e][4 * i + r, sl])
                    return c

                lax.fori_loop(0, CHUNK // 4, row_body, 0)
                pltpu.async_copy(
                    xv[s4x], out_hbm.at[pl.ds(rb, CHUNK)], so[s4x])

        issue_idx(0, 0)
        issue_idx(1, 1)
        issue_idx(2, 2)
        issue_idx(3, 3)
        issue_xg(0, 0, 0, 0)
        issue_xg(1, 1, 1, 1)

        def body12(g, carry):
            for dm in range(12):
                m = g * 12 + dm
                issue_xg(m + 2, (dm + 2) % 4, (dm + 2) % 4, (dm + 2) % 3)
                crunch(m, dm % 4, dm % 3)
                issue_idx(m + 4, dm % 4)
            return carry

        per_w_max = -(-n_chunks // nw)
        lax.fori_loop(0, (per_w_max + 11) // 12, body12, 0)

        for s in range(4):
            pltpu.make_async_copy(
                xv[s], out_hbm.at[pl.ds(0, CHUNK)], so[s]).wait()

    return sc_fn


def kernel(x, t, emb_table, W, b):
    p_table = pl.pallas_call(
        _proj_table_body,
        out_shape=jax.ShapeDtypeStruct((MAX_LEN, N_HID), jnp.float32),
    )(emb_table, W, b.reshape(1, N_HID))
    return _make_sc_kernel(x.shape[0])(x, t, p_table)


# speedup vs baseline: 1.0147x; 1.0147x over previous
import functools

import jax
import jax.numpy as jnp
from jax import lax
from jax.experimental import pallas as pl
from jax.experimental.pallas import tpu as pltpu
from jax.experimental.pallas import tpu_sc as plsc

N_HID = 128
MAX_LEN = 240
LANES = 16
CHUNK = 128
MAX_PW = 79


def _proj_table_body(emb_ref, w_ref, b_ref, out_ref):
    p = lax.dot_general(
        emb_ref[...], w_ref[...],
        dimension_numbers=(((1,), (1,)), ((), ())),
        preferred_element_type=jnp.float32,
    )
    out_ref[...] = p + b_ref[...]


@functools.cache
def _make_sc_kernel(n_rows):
    n_chunks = n_rows // CHUNK
    info = plsc.get_sparse_core_info()
    nc, ns = info.num_cores, info.num_subcores
    nw = nc * ns

    mesh = plsc.VectorSubcoreMesh(core_axis_name="c", subcore_axis_name="s")

    @functools.partial(
        pl.kernel,
        mesh=mesh,
        out_type=jax.ShapeDtypeStruct((n_rows, N_HID), jnp.float32),
        scratch_types=(
            [pltpu.VMEM((CHUNK,), jnp.int32) for _ in range(4)]
            + [pltpu.VMEM((CHUNK, N_HID), jnp.float32) for _ in range(4)]
            + [pltpu.VMEM((CHUNK, N_HID), jnp.float32) for _ in range(3)]
            + [pltpu.VMEM_SHARED((MAX_LEN, N_HID), jnp.float32)]
            + [pltpu.SemaphoreType.DMA for _ in range(15)]
        ),
    )
    def sc_fn(x_hbm, t_hbm, p_hbm, out_hbm,
              iv0, iv1, iv2, iv3, xv0, xv1, xv2, xv3, ev0, ev1, ev2, p_sh,
              si0, si1, si2, si3, sx0, sx1, sx2, sx3,
              sg0, sg1, sg2, so0, so1, so2, so3):
        wid = lax.axis_index("s") * nc + lax.axis_index("c")

        iv = (iv0, iv1, iv2, iv3)
        si = (si0, si1, si2, si3)
        xv = (xv0, xv1, xv2, xv3)
        sx = (sx0, sx1, sx2, sx3)
        so = (so0, so1, so2, so3)
        ev = (ev0, ev1, ev2)
        sg = (sg0, sg1, sg2)

        @pl.when(lax.axis_index("s") == 0)
        def _():
            pltpu.sync_copy(p_hbm, p_sh)
        plsc.subcore_barrier()

        def valid(m):
            return wid + m * nw < n_chunks

        def row_base(m):
            return (wid + m * nw) * CHUNK

        def issue_idx(m, s4):
            @pl.when(valid(m))
            def _():
                pltpu.async_copy(
                    t_hbm.at[pl.ds(row_base(m), CHUNK)], iv[s4], si[s4])

        def issue_xg(m, s4, s4x, s3e):
            @pl.when(valid(m))
            def _():
                pltpu.make_async_copy(
                    t_hbm.at[pl.ds(row_base(m), CHUNK)], iv[s4], si[s4]).wait()

                @pl.when(m >= 4)
                def _():
                    pltpu.make_async_copy(
                        xv[s4x], out_hbm.at[pl.ds(0, CHUNK)], so[s4x]).wait()

                pltpu.async_copy(
                    x_hbm.at[pl.ds(row_base(m), CHUNK)], xv[s4x], sx[s4x])
                pltpu.async_copy(p_sh.at[iv[s4]], ev[s3e], sg[s3e])

        def crunch(m, s4x, s3e):
            @pl.when(valid(m))
            def _():
                rb = row_base(m)
                pltpu.make_async_copy(
                    x_hbm.at[pl.ds(rb, CHUNK)], xv[s4x], sx[s4x]).wait()
                pltpu.make_async_copy(
                    p_sh.at[iv[s4x]], ev[s3e], sg[s3e]).wait()

                def row_body(i, c):
                    for r in range(2):
                        for j in range(N_HID // LANES):
                            sl = pl.ds(j * LANES, LANES)
                            plsc.addupdate(xv[s4x].at[2 * i + r, sl],
                                           ev[s3e][2 * i + r, sl])
                    return c

                lax.fori_loop(0, CHUNK // 2, row_body, 0)
                pltpu.async_copy(
                    xv[s4x], out_hbm.at[pl.ds(rb, CHUNK)], so[s4x])

        issue_idx(0, 0)
        issue_idx(1, 1)
        issue_idx(2, 2)
        issue_idx(3, 3)
        issue_xg(0, 0, 0, 0)
        issue_xg(1, 1, 1, 1)

        def body12(g, carry):
            for dm in range(12):
                m = g * 12 + dm
                issue_xg(m + 2, (dm + 2) % 4, (dm + 2) % 4, (dm + 2) % 3)
                crunch(m, dm % 4, dm % 3)
                issue_idx(m + 4, dm % 4)
            return carry

        per_w_max = -(-n_chunks // nw)
        lax.fori_loop(0, (per_w_max + 11) // 12, body12, 0)

        for s in range(4):
            pltpu.make_async_copy(
                xv[s], out_hbm.at[pl.ds(0, CHUNK)], so[s]).wait()

    return sc_fn


def kernel(x, t, emb_table, W, b):
    p_table = pl.pallas_call(
        _proj_table_body,
        out_shape=jax.ShapeDtypeStruct((MAX_LEN, N_HID), jnp.float32),
    )(emb_table, W, b.reshape(1, N_HID))
    return _make_sc_kernel(x.shape[0])(x, t, p_table)
